# baseline (speedup 1.0000x reference)
import jax
import jax.numpy as jnp
from jax.experimental import pallas as pl

FB = 512
NQ = 3
EPS = 1e-5
TILE = 256

_DNT = (((1,), (1,)), ((), ()))


def _mish(x):
    return x * jnp.tanh(jax.nn.softplus(x))


def _mm(x, w):
    return jax.lax.dot_general(x, w, _DNT, preferred_element_type=jnp.float32)


def _enc_kernel(x_ref, w00, b00, w01, b01, w02, b02,
                w10, b10, w11, b11, w12, b12,
                w20, b20, w21, b21, w22, b22,
                afx, prm, base, lth, dta, out_ref):
    x = x_ref[...]
    s0, c0 = prm[0, 0], prm[0, 1]
    s1, c1 = prm[0, 2], prm[0, 3]
    s2, c2 = prm[0, 4], prm[0, 5]
    h0 = jax.nn.relu(_mm(x, w00[...]) + b00[...])
    h1 = jax.nn.relu(_mm(h0[:, 32:], w01[...]) + b01[...])
    h2 = jax.nn.relu(_mm(h1[:, 21:], w02[...]) + b02[...])
    y = jnp.concatenate([h0[:, :32], h1[:, :21], h2], axis=1)
    y = _mish(y) * s0 + c0
    h0 = jax.nn.relu(_mm(y, w10[...]) + b10[...])
    h1 = jax.nn.relu(_mm(h0[:, 40:], w11[...]) + b11[...])
    h2 = jax.nn.relu(_mm(h1[:, 26:], w12[...]) + b12[...])
    y = jnp.concatenate([h0[:, :40], h1[:, :26], h2], axis=1)
    y = _mish(y) * s1 + c1
    h0 = jax.nn.relu(_mm(y, w20[...]) + b20[...])
    h1 = jax.nn.relu(_mm(h0[:, 16:], w21[...]) + b21[...])
    h2 = jax.nn.relu(_mm(h1[:, 10:], w22[...]) + b22[...])
    y = jnp.concatenate([h0[:, :16], h1[:, :10], h2], axis=1)
    y = _mish(y) * s2 + c2
    u = jnp.dot(y, afx[...], preferred_element_type=jnp.float32)
    bits = jnp.broadcast_to(base[0, :][None, :], u.shape)
    for i in range(7):
        m = u > lth[i, :][None, :]
        bits = jnp.where(m, bits + dta[i, :][None, :], bits)
    out_ref[...] = bits


def _vq_tables(cb, bff):
    v = [cb[:, k] for k in range(8)]
    ix = [jnp.full((FB,), k, jnp.float32) for k in range(8)]
    for rnd in range(8):
        for i in range(rnd & 1, 7, 2):
            sw = v[i + 1] < v[i]
            lo = jnp.where(sw, v[i + 1], v[i])
            hi = jnp.where(sw, v[i], v[i + 1])
            jlo = jnp.where(sw, ix[i + 1], ix[i])
            jhi = jnp.where(sw, ix[i], ix[i + 1])
            v[i], v[i + 1], ix[i], ix[i + 1] = lo, hi, jlo, jhi
    t = [0.5 * (v[i] + v[i + 1]) for i in range(7)]
    L = jnp.stack([jnp.log(ti) - jnp.log1p(-ti) for ti in t])
    order = jnp.stack(ix).astype(jnp.int32)
    bits3 = jnp.stack([(order >> 2) & 1, (order >> 1) & 1, order & 1],
                      axis=2).astype(jnp.float32)
    base = bits3[0].reshape(1, FB * NQ)
    dta = (bits3[1:] - bits3[:-1]).reshape(7, FB * NQ)
    lth = jnp.repeat(L, NQ, axis=1).reshape(7, FB * NQ) - bff[None, :]
    return base, lth, dta


def kernel(x, W00, b00, W01, b01, W02, b02, W10, b10, W11, b11, W12, b12,
           W20, b20, W21, b21, W22, b22, Wf, bf,
           bn0_w, bn0_b, bn1_w, bn1_b, bn2_w, bn2_b, cb):
    Bsz = x.shape[0]
    xf = x.reshape(Bsz, -1)

    inv = 1.0 / jnp.sqrt(1.0 + EPS)
    prm = jnp.stack([bn0_w[0] * inv, bn0_b[0], bn1_w[0] * inv, bn1_b[0],
                     bn2_w[0] * inv, bn2_b[0], bn2_b[0], bn2_b[0]]
                    ).reshape(1, 8)

    afx = jnp.repeat(Wf.T, NQ, axis=1)
    bff = jnp.repeat(bf, NQ, axis=0)
    base, lth, dta = _vq_tables(cb.reshape(FB, 8), bff)

    grid = (Bsz // TILE,)
    row = lambda i: (i, 0)
    fixed = lambda i: (0, 0)

    def wspec(a):
        return pl.BlockSpec(a.shape, fixed)

    weights = [W00, W01, W02, W10, W11, W12, W20, W21, W22]
    biases = [b00.reshape(1, -1), b01.reshape(1, -1), b02.reshape(1, -1),
              b10.reshape(1, -1), b11.reshape(1, -1), b12.reshape(1, -1),
              b20.reshape(1, -1), b21.reshape(1, -1), b22.reshape(1, -1)]
    operands = [xf]
    for w, b in zip(weights, biases):
        operands += [w, b]
    operands += [afx, prm, base, lth, dta]

    in_specs = [pl.BlockSpec((TILE, 768), row)]
    for w, b in zip(weights, biases):
        in_specs += [wspec(w), wspec(b)]
    in_specs += [wspec(afx), wspec(prm), wspec(base), wspec(lth), wspec(dta)]

    out = pl.pallas_call(
        _enc_kernel,
        grid=grid,
        in_specs=in_specs,
        out_specs=pl.BlockSpec((TILE, FB * NQ), row),
        out_shape=jax.ShapeDtypeStruct((Bsz, FB * NQ), jnp.float32),
    )(*operands)
    return out


# speedup vs baseline: 1.2980x; 1.2980x over previous
import jax
import jax.numpy as jnp
from jax.experimental import pallas as pl

FB = 512
NQ = 3
EPS = 1e-5
TILE = 1024

_DNT = (((1,), (1,)), ((), ()))


def _mish(x):
    return x * jnp.tanh(jax.nn.softplus(x))


def _mm(x, w):
    return jax.lax.dot_general(x, w, _DNT, preferred_element_type=jnp.float32)


def _enc_kernel(x_ref, w00, b00, w01, b01, w02, b02,
                w10, b10, w11, b11, w12, b12,
                w20, b20, w21, b21, w22, b22,
                afx, prm, base, lth, dta, out_ref):
    x = x_ref[...]
    s0, c0 = prm[0, 0], prm[0, 1]
    s1, c1 = prm[0, 2], prm[0, 3]
    s2, c2 = prm[0, 4], prm[0, 5]
    h0 = jax.nn.relu(_mm(x, w00[...]) + b00[...])
    h1 = jax.nn.relu(_mm(h0[:, 32:], w01[...]) + b01[...])
    h2 = jax.nn.relu(_mm(h1[:, 21:], w02[...]) + b02[...])
    y = jnp.concatenate([h0[:, :32], h1[:, :21], h2], axis=1)
    y = _mish(y) * s0 + c0
    h0 = jax.nn.relu(_mm(y, w10[...]) + b10[...])
    h1 = jax.nn.relu(_mm(h0[:, 40:], w11[...]) + b11[...])
    h2 = jax.nn.relu(_mm(h1[:, 26:], w12[...]) + b12[...])
    y = jnp.concatenate([h0[:, :40], h1[:, :26], h2], axis=1)
    y = _mish(y) * s1 + c1
    h0 = jax.nn.relu(_mm(y, w20[...]) + b20[...])
    h1 = jax.nn.relu(_mm(h0[:, 16:], w21[...]) + b21[...])
    h2 = jax.nn.relu(_mm(h1[:, 10:], w22[...]) + b22[...])
    y = jnp.concatenate([h0[:, :16], h1[:, :10], h2], axis=1)
    y = _mish(y) * s2 + c2
    u = jnp.dot(y, afx[...], preferred_element_type=jnp.float32)
    bits = jnp.broadcast_to(base[0, :][None, :], u.shape)
    for i in range(7):
        m = u > lth[i, :][None, :]
        bits = jnp.where(m, bits + dta[i, :][None, :], bits)
    out_ref[...] = bits


def _vq_tables(cb, bff):
    v = [cb[:, k] for k in range(8)]
    ix = [jnp.full((FB,), k, jnp.float32) for k in range(8)]
    for rnd in range(8):
        for i in range(rnd & 1, 7, 2):
            sw = v[i + 1] < v[i]
            lo = jnp.where(sw, v[i + 1], v[i])
            hi = jnp.where(sw, v[i], v[i + 1])
            jlo = jnp.where(sw, ix[i + 1], ix[i])
            jhi = jnp.where(sw, ix[i], ix[i + 1])
            v[i], v[i + 1], ix[i], ix[i + 1] = lo, hi, jlo, jhi
    t = [0.5 * (v[i] + v[i + 1]) for i in range(7)]
    L = jnp.stack([jnp.log(ti) - jnp.log1p(-ti) for ti in t])
    order = jnp.stack(ix).astype(jnp.int32)
    bits3 = jnp.stack([(order >> 2) & 1, (order >> 1) & 1, order & 1],
                      axis=2).astype(jnp.float32)
    base = bits3[0].reshape(1, FB * NQ)
    dta = (bits3[1:] - bits3[:-1]).reshape(7, FB * NQ)
    lth = jnp.repeat(L, NQ, axis=1).reshape(7, FB * NQ) - bff[None, :]
    return base, lth, dta


def kernel(x, W00, b00, W01, b01, W02, b02, W10, b10, W11, b11, W12, b12,
           W20, b20, W21, b21, W22, b22, Wf, bf,
           bn0_w, bn0_b, bn1_w, bn1_b, bn2_w, bn2_b, cb):
    Bsz = x.shape[0]
    xf = x.reshape(Bsz, -1)

    inv = 1.0 / jnp.sqrt(1.0 + EPS)
    prm = jnp.stack([bn0_w[0] * inv, bn0_b[0], bn1_w[0] * inv, bn1_b[0],
                     bn2_w[0] * inv, bn2_b[0], bn2_b[0], bn2_b[0]]
                    ).reshape(1, 8)

    afx = jnp.repeat(Wf.T, NQ, axis=1)
    bff = jnp.repeat(bf, NQ, axis=0)
    base, lth, dta = _vq_tables(cb.reshape(FB, 8), bff)

    grid = (Bsz // TILE,)
    row = lambda i: (i, 0)
    fixed = lambda i: (0, 0)

    def wspec(a):
        return pl.BlockSpec(a.shape, fixed)

    weights = [W00, W01, W02, W10, W11, W12, W20, W21, W22]
    biases = [b00.reshape(1, -1), b01.reshape(1, -1), b02.reshape(1, -1),
              b10.reshape(1, -1), b11.reshape(1, -1), b12.reshape(1, -1),
              b20.reshape(1, -1), b21.reshape(1, -1), b22.reshape(1, -1)]
    operands = [xf]
    for w, b in zip(weights, biases):
        operands += [w, b]
    operands += [afx, prm, base, lth, dta]

    in_specs = [pl.BlockSpec((TILE, 768), row)]
    for w, b in zip(weights, biases):
        in_specs += [wspec(w), wspec(b)]
    in_specs += [wspec(afx), wspec(prm), wspec(base), wspec(lth), wspec(dta)]

    out = pl.pallas_call(
        _enc_kernel,
        grid=grid,
        in_specs=in_specs,
        out_specs=pl.BlockSpec((TILE, FB * NQ), row),
        out_shape=jax.ShapeDtypeStruct((Bsz, FB * NQ), jnp.float32),
    )(*operands)
    return out
